# baseline (speedup 1.0000x reference)
import jax
import jax.numpy as jnp
from jax import lax
from jax.experimental import pallas as pl
from jax.experimental.pallas import tpu as pltpu
from jax.experimental.pallas import tpu_sc as plsc
from jax._src.pallas import mpmd

S = 64
K = 256
L = 16

_SMESH = plsc.ScalarSubcoreMesh(axis_name="c", num_cores=1)
_VMESH = plsc.VectorSubcoreMesh(
    core_axis_name="c", subcore_axis_name="s", num_cores=1, num_subcores=1)


def _scs_fn(x_hbm, idx_hbm, vals_hbm, out_hbm,
            x_sp, idx_sp, vals_sp, scs_sem, rdy):
    cp_x = pltpu.make_async_copy(x_hbm, x_sp, scs_sem)
    cp_i = pltpu.make_async_copy(idx_hbm, idx_sp, scs_sem)
    cp_v = pltpu.make_async_copy(vals_hbm, vals_sp, scs_sem)
    cp_x.start()
    cp_i.start()
    cp_v.start()
    cp_x.wait()
    cp_i.wait()
    cp_v.wait()
    pl.semaphore_signal(
        rdy, 1, device_id={"c": 0, "s": 0},
        device_id_type=pl.DeviceIdType.MESH)


def _tec_fn(x_hbm, idx_hbm, vals_hbm, out_hbm,
            x_sp, idx_sp, vals_sp, scs_sem, rdy):
    def scoped(x_v, idx_v, vals_v,
               acc_a, acc_b, acc_c, acc_d,
               acc_e, acc_f, acc_g, acc_h, tec_sem):
        accs = (acc_a, acc_b, acc_c, acc_d, acc_e, acc_f, acc_g, acc_h)
        zero = jnp.zeros((L,), jnp.float32)
        for j in range(S // L):
            for acc in accs:
                acc[pl.ds(j * L, L)] = zero

        pl.semaphore_wait(rdy, 1)

        cp_x = pltpu.make_async_copy(x_sp, x_v, tec_sem)
        cp_i = pltpu.make_async_copy(idx_sp, idx_v, tec_sem)
        cp_v = pltpu.make_async_copy(vals_sp, vals_v, tec_sem)
        cp_x.start()
        cp_i.start()
        cp_v.start()
        cp_x.wait()
        cp_i.wait()
        cp_v.wait()

        rs = []
        gs = []
        for i in range(K // L):
            r = idx_v[0, pl.ds(i * L, L)]
            c = idx_v[1, pl.ds(i * L, L)]
            v = vals_v[pl.ds(i * L, L)]
            rs.append(r)
            gs.append(plsc.load_gather(x_v, [c]) * v)

        for i in range(K // L):
            plsc.addupdate_scatter(accs[i % 8], [rs[i]], gs[i])

        for j in range(S // L):
            sl = pl.ds(j * L, L)
            acc_a[sl] = (
                ((acc_a[sl] + acc_b[sl]) + (acc_c[sl] + acc_d[sl]))
                + ((acc_e[sl] + acc_f[sl]) + (acc_g[sl] + acc_h[sl]))
            )

        pltpu.sync_copy(acc_a, out_hbm)

    pl.run_scoped(
        scoped,
        pltpu.VMEM((S,), jnp.float32),
        pltpu.VMEM((2, K), jnp.int32),
        pltpu.VMEM((K,), jnp.float32),
        pltpu.VMEM((S,), jnp.float32),
        pltpu.VMEM((S,), jnp.float32),
        pltpu.VMEM((S,), jnp.float32),
        pltpu.VMEM((S,), jnp.float32),
        pltpu.VMEM((S,), jnp.float32),
        pltpu.VMEM((S,), jnp.float32),
        pltpu.VMEM((S,), jnp.float32),
        pltpu.VMEM((S,), jnp.float32),
        pltpu.SemaphoreType.DMA,
    )


@jax.jit
def _spmv(x, idx, vals):
    return mpmd.mpmd_map(
        [(_SMESH, _scs_fn), (_VMESH, _tec_fn)],
        out_types=[jax.ShapeDtypeStruct((S,), jnp.float32)],
        scratch_types=[
            pltpu.VMEM_SHARED((S,), jnp.float32),
            pltpu.VMEM_SHARED((2, K), jnp.int32),
            pltpu.VMEM_SHARED((K,), jnp.float32),
            pltpu.SemaphoreType.DMA @ _SMESH,
            pltpu.SemaphoreType.REGULAR @ _VMESH,
        ],
        compiler_params=pltpu.CompilerParams(needs_layout_passes=False),
    )(x, idx, vals)[0]


def kernel(x, indices, values):
    return _spmv(x, indices.astype(jnp.int32), values)


# speedup vs baseline: 1.0057x; 1.0057x over previous
import jax
import jax.numpy as jnp
from jax import lax
from jax.experimental import pallas as pl
from jax.experimental.pallas import tpu as pltpu
from jax.experimental.pallas import tpu_sc as plsc
from jax._src.pallas import mpmd

S = 64
K = 256
L = 16

_SMESH = plsc.ScalarSubcoreMesh(axis_name="c", num_cores=1)
_VMESH = plsc.VectorSubcoreMesh(
    core_axis_name="c", subcore_axis_name="s", num_cores=1, num_subcores=1)


def _scs_fn(x_hbm, idx_hbm, vals_hbm, out_hbm,
            x_sp, idx_sp, vals_sp, scs_sem, rdy):
    cp_x = pltpu.make_async_copy(x_hbm, x_sp, scs_sem)
    cp_i = pltpu.make_async_copy(idx_hbm, idx_sp, scs_sem)
    cp_v = pltpu.make_async_copy(vals_hbm, vals_sp, scs_sem)
    cp_x.start()
    cp_i.start()
    cp_v.start()
    cp_x.wait()
    cp_i.wait()
    cp_v.wait()
    pl.semaphore_signal(
        rdy, 1, device_id={"c": 0, "s": 0},
        device_id_type=pl.DeviceIdType.MESH)


def _tec_fn(x_hbm, idx_hbm, vals_hbm, out_hbm,
            x_sp, idx_sp, vals_sp, scs_sem, rdy):
    def scoped(x_v, idx_v, vals_v, acc_a, acc_b, acc_c, acc_d, tec_sem):
        accs = (acc_a, acc_b, acc_c, acc_d)
        zero = jnp.zeros((L,), jnp.float32)
        for j in range(S // L):
            for acc in accs:
                acc[pl.ds(j * L, L)] = zero

        pl.semaphore_wait(rdy, 1)

        cp_x = pltpu.make_async_copy(x_sp, x_v, tec_sem)
        cp_i = pltpu.make_async_copy(idx_sp, idx_v, tec_sem)
        cp_v = pltpu.make_async_copy(vals_sp, vals_v, tec_sem)
        cp_x.start()
        cp_i.start()
        cp_v.start()
        cp_x.wait()
        cp_i.wait()
        cp_v.wait()

        rs = []
        gs = []
        for i in range(K // L):
            r = idx_v[0, pl.ds(i * L, L)]
            c = idx_v[1, pl.ds(i * L, L)]
            v = vals_v[pl.ds(i * L, L)]
            rs.append(r)
            gs.append(plsc.load_gather(x_v, [c]) * v)

        for i in range(K // L):
            plsc.addupdate_scatter(accs[i % 4], [rs[i]], gs[i])

        for j in range(S // L):
            sl = pl.ds(j * L, L)
            acc_a[sl] = (acc_a[sl] + acc_b[sl]) + (acc_c[sl] + acc_d[sl])

        pltpu.sync_copy(acc_a, out_hbm)

    pl.run_scoped(
        scoped,
        pltpu.VMEM((S,), jnp.float32),
        pltpu.VMEM((2, K), jnp.int32),
        pltpu.VMEM((K,), jnp.float32),
        pltpu.VMEM((S,), jnp.float32),
        pltpu.VMEM((S,), jnp.float32),
        pltpu.VMEM((S,), jnp.float32),
        pltpu.VMEM((S,), jnp.float32),
        pltpu.SemaphoreType.DMA,
    )


@jax.jit
def _spmv(x, idx, vals):
    return mpmd.mpmd_map(
        [(_SMESH, _scs_fn), (_VMESH, _tec_fn)],
        out_types=[jax.ShapeDtypeStruct((S,), jnp.float32)],
        scratch_types=[
            pltpu.VMEM_SHARED((S,), jnp.float32),
            pltpu.VMEM_SHARED((2, K), jnp.int32),
            pltpu.VMEM_SHARED((K,), jnp.float32),
            pltpu.SemaphoreType.DMA @ _SMESH,
            pltpu.SemaphoreType.REGULAR @ _VMESH,
        ],
        compiler_params=pltpu.CompilerParams(needs_layout_passes=False),
    )(x, idx, vals)[0]


def kernel(x, indices, values):
    return _spmv(x, indices.astype(jnp.int32), values)
